# baseline (speedup 1.0000x reference)
import functools

import jax
import jax.numpy as jnp
from jax import lax
from jax.experimental import pallas as pl
from jax.experimental.pallas import tpu as pltpu
from jax.experimental.pallas import tpu_sc as plsc

_N = 1000000
_K = 16
_B = 16384

try:
    _info = plsc.get_sparse_core_info()
    _NC, _NS = _info.num_cores, _info.num_subcores
except Exception:
    _NC, _NS = 2, 16
_NW = _NC * _NS
_BPW = _B // _NW

_mesh = plsc.VectorSubcoreMesh(core_axis_name="c", subcore_axis_name="s")


@functools.partial(
    pl.kernel,
    mesh=_mesh,
    out_type=(
        jax.ShapeDtypeStruct((_K * _B,), jnp.float32),
        jax.ShapeDtypeStruct((_B,), jnp.float32),
    ),
    scratch_types=[
        pltpu.VMEM((_BPW,), jnp.int32),
        pltpu.VMEM((_K * _BPW,), jnp.int32),
        pltpu.VMEM((_K * _BPW,), jnp.float32),
        pltpu.VMEM((_BPW,), jnp.float32),
        pltpu.SemaphoreType.DMA,
        pltpu.SemaphoreType.DMA,
    ],
    compiler_params=pltpu.CompilerParams(use_tc_tiling_on_sc=False,
                                         skip_device_barrier=True),
)
def _gather_kernel(idx_hbm, pos_hbm, het_hbm, out_pos, out_het,
                   idx_v, wrd_v, pos_v, het_v, sem_p, sem_h):
    wid = lax.axis_index("s") * _NC + lax.axis_index("c")
    base = wid * _BPW
    pltpu.sync_copy(idx_hbm.at[pl.ds(base, _BPW)], idx_v)

    cp_h = pltpu.async_copy(het_hbm.at[idx_v], het_v, sem_h)

    def wrd_body(g):
        v = jax.lax.shift_left(idx_v[pl.ds(g * 16, 16)], 4)
        for k in range(_K):
            wrd_v[pl.ds(k * _BPW + g * 16, 16)] = v + k

    pl.loop(0, _BPW // 16)(wrd_body)

    copies = []
    for k in range(_K):
        copies.append(
            pltpu.async_copy(pos_hbm.at[wrd_v.at[pl.ds(k * _BPW, _BPW)]],
                             pos_v.at[pl.ds(k * _BPW, _BPW)], sem_p))
    for cp in copies:
        cp.wait()
    cp_h.wait()

    for k in range(_K):
        pltpu.sync_copy(pos_v.at[pl.ds(k * _BPW, _BPW)],
                        out_pos.at[pl.ds(k * _B + base, _BPW)])
    pltpu.sync_copy(het_v, out_het.at[pl.ds(base, _BPW)])


def kernel(indices, values_pos, values_het):
    idx = indices.astype(jnp.int32)
    tail = jnp.zeros((1024,), jnp.float32) + (indices[0] * 0).astype(
        jnp.float32)
    pos_flat = jnp.concatenate([values_pos.reshape(-1), tail])
    pos_kb, het_flat = _gather_kernel(idx, pos_flat, values_het.reshape(-1))
    return (pos_kb.reshape(_K, _B).T, het_flat.reshape(_B, 1))


# speedup vs baseline: 1.0876x; 1.0876x over previous
import functools

import jax
import jax.numpy as jnp
from jax import lax
from jax.experimental import pallas as pl
from jax.experimental.pallas import tpu as pltpu
from jax.experimental.pallas import tpu_sc as plsc

_N = 1000000
_K = 16
_B = 16384

try:
    _info = plsc.get_sparse_core_info()
    _NC, _NS = _info.num_cores, _info.num_subcores
except Exception:
    _NC, _NS = 2, 16
_NW = _NC * _NS
_BPW = _B // _NW

_mesh = plsc.VectorSubcoreMesh(core_axis_name="c", subcore_axis_name="s")


@functools.partial(
    pl.kernel,
    mesh=_mesh,
    out_type=(
        jax.ShapeDtypeStruct((_B, _K), jnp.float32),
        jax.ShapeDtypeStruct((_B,), jnp.float32),
    ),
    scratch_types=[
        pltpu.VMEM((_BPW,), jnp.int32),
        pltpu.VMEM((_BPW, _K), jnp.float32),
        pltpu.VMEM((_BPW,), jnp.float32),
        pltpu.SemaphoreType.DMA,
        pltpu.SemaphoreType.DMA,
    ],
    compiler_params=pltpu.CompilerParams(use_tc_tiling_on_sc=False,
                                         skip_device_barrier=True),
)
def _gather_kernel(idx_hbm, pos_hbm, het_hbm, out_pos, out_het,
                   idx_v, pos_v, het_v, sem_p, sem_h):
    wid = lax.axis_index("s") * _NC + lax.axis_index("c")
    base = wid * _BPW
    pltpu.sync_copy(idx_hbm.at[pl.ds(base, _BPW)], idx_v)
    cp_p = pltpu.async_copy(pos_hbm.at[idx_v], pos_v, sem_p)
    cp_h = pltpu.async_copy(het_hbm.at[idx_v], het_v, sem_h)
    cp_p.wait()
    cp_h.wait()
    pltpu.sync_copy(pos_v, out_pos.at[pl.ds(base, _BPW)])
    pltpu.sync_copy(het_v, out_het.at[pl.ds(base, _BPW)])


def kernel(indices, values_pos, values_het):
    idx = indices.astype(jnp.int32)
    pos, het_flat = _gather_kernel(idx, values_pos, values_het.reshape(-1))
    return (pos, het_flat.reshape(_B, 1))


# speedup vs baseline: 4.4454x; 4.0872x over previous
import functools

import jax
import jax.numpy as jnp
from jax import lax
from jax.experimental import pallas as pl
from jax.experimental.pallas import tpu as pltpu
from jax.experimental.pallas import tpu_sc as plsc

_N = 1000000
_K = 16
_B = 16384
_NBLK = 7813
_HALF = _NBLK * 1024

try:
    _info = plsc.get_sparse_core_info()
    _NC, _NS = _info.num_cores, _info.num_subcores
except Exception:
    _NC, _NS = 2, 16
_NW = _NC * _NS
_BPW = _B // _NW

_mesh = plsc.VectorSubcoreMesh(core_axis_name="c", subcore_axis_name="s")


@functools.partial(
    pl.kernel,
    mesh=_mesh,
    out_type=(
        jax.ShapeDtypeStruct((_K * _B,), jnp.float32),
        jax.ShapeDtypeStruct((_B,), jnp.float32),
    ),
    scratch_types=[
        pltpu.VMEM((_BPW,), jnp.int32),
        pltpu.VMEM((_BPW,), jnp.int32),
        pltpu.VMEM((_K * _BPW,), jnp.float32),
        pltpu.VMEM((_BPW,), jnp.float32),
        pltpu.SemaphoreType.DMA,
        pltpu.SemaphoreType.DMA,
    ],
    compiler_params=pltpu.CompilerParams(use_tc_tiling_on_sc=False,
                                         skip_device_barrier=True),
)
def _gather_kernel(idx_hbm, pos_hbm, het_hbm, out_pos, out_het,
                   idx_v, wrd_v, pos_v, het_v, sem_p, sem_h):
    wid = lax.axis_index("s") * _NC + lax.axis_index("c")
    base = wid * _BPW
    pltpu.sync_copy(idx_hbm.at[pl.ds(base, _BPW)], idx_v)

    cp_h = pltpu.async_copy(het_hbm.at[idx_v], het_v, sem_h)

    def wrd_body(g):
        v = idx_v[pl.ds(g * 16, 16)]
        wrd_v[pl.ds(g * 16, 16)] = (
            jax.lax.shift_left(jax.lax.shift_right_logical(v, 7), 10)
            | (v & 127))

    pl.loop(0, _BPW // 16)(wrd_body)

    copies = []
    for k in range(_K):
        off = (k // 8) * _HALF + (k % 8) * 128
        src = pos_hbm.at[pl.ds(off, _HALF * 2 - off)]
        copies.append(
            pltpu.async_copy(src.at[wrd_v],
                             pos_v.at[pl.ds(k * _BPW, _BPW)], sem_p))
    for cp in copies:
        cp.wait()
    cp_h.wait()

    for k in range(_K):
        pltpu.sync_copy(pos_v.at[pl.ds(k * _BPW, _BPW)],
                        out_pos.at[pl.ds(k * _B + base, _BPW)])
    pltpu.sync_copy(het_v, out_het.at[pl.ds(base, _BPW)])


def kernel(indices, values_pos, values_het):
    idx = indices.astype(jnp.int32)
    pos_y = (jnp.pad(values_pos, ((0, 64), (0, 0)))
             .reshape(_NBLK, 128, 2, 8)
             .transpose(2, 0, 3, 1)
             .reshape(-1))
    pos_kb, het_flat = _gather_kernel(idx, pos_y, values_het.reshape(-1))
    return (pos_kb.reshape(_K, _B).T, het_flat.reshape(_B, 1))
